# baseline (speedup 1.0000x reference)
import functools

import jax
import jax.numpy as jnp
from jax import lax
from jax.experimental import pallas as pl
from jax.experimental.pallas import tpu as pltpu
from jax.experimental.pallas import tpu_sc as plsc

B, J, O, M, D = 64, 50, 50, 50, 128
JP = 64
_PAD = 1e30


def _sc_body(tjr_hbm, tma_hbm, jnm_hbm, sh_hbm, ash_hbm,
             tjr_v, tma_v, jnm_v, sh_v, ash_v):
    c = lax.axis_index("c")
    s = lax.axis_index("s")
    wid = s * 2 + c
    for r in range(B // 32):
        b = wid * (B // 32) + r
        pltpu.sync_copy(tjr_hbm.at[b], tjr_v)
        pltpu.sync_copy(tma_hbm.at[b], tma_v)
        pltpu.sync_copy(jnm_hbm.at[b], jnm_v)
        scheds = []
        tmas = []
        for ch in range(JP // 16):
            idx = jnm_v[pl.ds(ch * 16, 16)]
            g = plsc.load_gather(tma_v, [idx])
            scheds.append(jnp.maximum(tjr_v[pl.ds(ch * 16, 16)], g))
            tmas.append(tma_v[pl.ds(ch * 16, 16)])
        m1 = jnp.minimum(jnp.minimum(scheds[0], scheds[1]),
                         jnp.minimum(scheds[2], scheds[3]))
        mn1 = jnp.min(m1)
        m2 = jnp.minimum(jnp.minimum(tmas[0], tmas[1]),
                         jnp.minimum(tmas[2], tmas[3]))
        mn2 = jnp.min(m2)
        for ch in range(JP // 16):
            sh_v[pl.ds(ch * 16, 16)] = (scheds[ch] - mn1) * 0.01
            ash_v[pl.ds(ch * 16, 16)] = tmas[ch] - mn2
        pltpu.sync_copy(sh_v, sh_hbm.at[b])
        pltpu.sync_copy(ash_v, ash_hbm.at[b])


def _sc_prep(tjr_p, tma_p, jnm_p):
    mesh = plsc.VectorSubcoreMesh(core_axis_name="c", subcore_axis_name="s")
    fn = functools.partial(
        pl.kernel,
        out_type=[jax.ShapeDtypeStruct((B, JP), jnp.float32),
                  jax.ShapeDtypeStruct((B, JP), jnp.float32)],
        mesh=mesh,
        compiler_params=pltpu.CompilerParams(needs_layout_passes=False),
        scratch_types=[
            pltpu.VMEM((JP,), jnp.float32),
            pltpu.VMEM((JP,), jnp.float32),
            pltpu.VMEM((JP,), jnp.int32),
            pltpu.VMEM((JP,), jnp.float32),
            pltpu.VMEM((JP,), jnp.float32),
        ],
    )(_sc_body)
    return fn(tjr_p, tma_p, jnm_p)


BB = 8


def _tc_body(proc_ref, nof_ref, shs_ref, ash_ref, w2_ref, wma_ref,
             ops_ref, ma_ref):
    f32 = jnp.float32
    d = lax.broadcasted_iota(jnp.int32, (1, D), 1).astype(f32)
    half = jnp.float32(D // 2)
    dm = jnp.where(d < half, d, d - half)
    freq = jnp.exp(dm * jnp.float32(-jnp.log(10000.0) / (D // 2)))
    phase = jnp.where(d < half, 0.0, jnp.pi / 2).astype(f32)
    o_col = lax.broadcasted_iota(jnp.int32, (O, 1), 0).astype(f32)
    ang_o = o_col * freq
    s_o = jnp.sin(ang_o)[None, None]
    c_o = jnp.cos(ang_o)[None, None]
    n = nof_ref[...]
    ang_n = n * freq[None] + phase[None]
    s_n = jnp.sin(ang_n)[:, :, None]
    c_n = jnp.cos(ang_n)[:, :, None]
    pe = s_o * c_n + c_o * s_n
    coef_a = proc_ref[...] * 0.01
    o_row = lax.broadcasted_iota(jnp.int32, (BB, J, O), 2).astype(f32)
    coef_b = jnp.where(n == o_row, shs_ref[...], 0.0)
    w0 = w2_ref[0:1, :][None, None]
    w1 = w2_ref[1:2, :][None, None]
    ops_ref[...] = coef_a[..., None] * w0 + coef_b[..., None] * w1 + pe
    ma_ref[...] = ash_ref[...] * wma_ref[0:1, :][None]


def _tc_embed(proc_times, nof, shs, ash, w2, wma):
    return pl.pallas_call(
        _tc_body,
        grid=(B // BB,),
        in_specs=[
            pl.BlockSpec((BB, J, O), lambda b: (b, 0, 0)),
            pl.BlockSpec((BB, J, 1), lambda b: (b, 0, 0)),
            pl.BlockSpec((BB, J, 1), lambda b: (b, 0, 0)),
            pl.BlockSpec((BB, M, 1), lambda b: (b, 0, 0)),
            pl.BlockSpec((2, D), lambda b: (0, 0)),
            pl.BlockSpec((1, D), lambda b: (0, 0)),
        ],
        out_specs=[
            pl.BlockSpec((BB, J, O, D), lambda b: (b, 0, 0, 0)),
            pl.BlockSpec((BB, M, D), lambda b: (b, 0, 0)),
        ],
        out_shape=[
            jax.ShapeDtypeStruct((B, J, O, D), jnp.float32),
            jax.ShapeDtypeStruct((B, M, D), jnp.float32),
        ],
    )(proc_times, nof, shs, ash, w2, wma)


def kernel(proc_times, time_job_ready, time_ma_ready, next_op, job_next_ma,
           W_ops, W_ma):
    tjr_p = jnp.pad(time_job_ready, ((0, 0), (0, JP - J)),
                    constant_values=_PAD)
    tma_p = jnp.pad(time_ma_ready, ((0, 0), (0, JP - M)),
                    constant_values=_PAD)
    jnm_p = jnp.pad(job_next_ma.astype(jnp.int32), ((0, 0), (0, JP - J)))
    sh_full, ash_full = _sc_prep(tjr_p, tma_p, jnm_p)
    shs = sh_full[:, :J].reshape(B, J, 1)
    ash = ash_full[:, :M].reshape(B, M, 1)
    nof = next_op.astype(jnp.float32).reshape(B, J, 1)
    w2 = W_ops.T
    wma = W_ma.T
    ops_emb, ma_emb = _tc_embed(proc_times, nof, shs, ash, w2, wma)
    return (ops_emb, ma_emb)


# speedup vs baseline: 1.0035x; 1.0035x over previous
import functools

import jax
import jax.numpy as jnp
from jax import lax
from jax.experimental import pallas as pl
from jax.experimental.pallas import tpu as pltpu
from jax.experimental.pallas import tpu_sc as plsc

B, J, O, M, D = 64, 50, 50, 50, 128
JP = 64
_PAD = 1e30


def _sc_body(tjr_hbm, tma_hbm, jnm_hbm, sh_hbm, ash_hbm,
             tjr_v, tma_v, jnm_v, sh_v, ash_v):
    c = lax.axis_index("c")
    s = lax.axis_index("s")
    wid = s * 2 + c
    for r in range(B // 32):
        b = wid * (B // 32) + r
        pltpu.sync_copy(tjr_hbm.at[b], tjr_v)
        pltpu.sync_copy(tma_hbm.at[b], tma_v)
        pltpu.sync_copy(jnm_hbm.at[b], jnm_v)
        scheds = []
        tmas = []
        for ch in range(JP // 16):
            idx = jnm_v[pl.ds(ch * 16, 16)]
            g = plsc.load_gather(tma_v, [idx])
            scheds.append(jnp.maximum(tjr_v[pl.ds(ch * 16, 16)], g))
            tmas.append(tma_v[pl.ds(ch * 16, 16)])
        m1 = jnp.minimum(jnp.minimum(scheds[0], scheds[1]),
                         jnp.minimum(scheds[2], scheds[3]))
        mn1 = jnp.min(m1)
        m2 = jnp.minimum(jnp.minimum(tmas[0], tmas[1]),
                         jnp.minimum(tmas[2], tmas[3]))
        mn2 = jnp.min(m2)
        for ch in range(JP // 16):
            sh_v[pl.ds(ch * 16, 16)] = (scheds[ch] - mn1) * 0.01
            ash_v[pl.ds(ch * 16, 16)] = tmas[ch] - mn2
        pltpu.sync_copy(sh_v, sh_hbm.at[b])
        pltpu.sync_copy(ash_v, ash_hbm.at[b])


def _sc_prep(tjr_p, tma_p, jnm_p):
    mesh = plsc.VectorSubcoreMesh(core_axis_name="c", subcore_axis_name="s")
    fn = functools.partial(
        pl.kernel,
        out_type=[jax.ShapeDtypeStruct((B, JP), jnp.float32),
                  jax.ShapeDtypeStruct((B, JP), jnp.float32)],
        mesh=mesh,
        compiler_params=pltpu.CompilerParams(needs_layout_passes=False),
        scratch_types=[
            pltpu.VMEM((JP,), jnp.float32),
            pltpu.VMEM((JP,), jnp.float32),
            pltpu.VMEM((JP,), jnp.int32),
            pltpu.VMEM((JP,), jnp.float32),
            pltpu.VMEM((JP,), jnp.float32),
        ],
    )(_sc_body)
    return fn(tjr_p, tma_p, jnm_p)


BB = 2


def _tc_body(proc_ref, nof_ref, shs_ref, ash_ref, w2_ref, wma_ref,
             ops_ref, ma_ref):
    f32 = jnp.float32
    d = lax.broadcasted_iota(jnp.int32, (1, D), 1).astype(f32)
    half = jnp.float32(D // 2)
    dm = jnp.where(d < half, d, d - half)
    freq = jnp.exp(dm * jnp.float32(-jnp.log(10000.0) / (D // 2)))
    phase = jnp.where(d < half, 0.0, jnp.pi / 2).astype(f32)
    o_col = lax.broadcasted_iota(jnp.int32, (O, 1), 0).astype(f32)
    ang_o = o_col * freq
    s_o = jnp.sin(ang_o)[None, None]
    c_o = jnp.cos(ang_o)[None, None]
    n = nof_ref[...]
    ang_n = n * freq[None] + phase[None]
    s_n = jnp.sin(ang_n)[:, :, None]
    c_n = jnp.cos(ang_n)[:, :, None]
    pe = s_o * c_n + c_o * s_n
    coef_a = proc_ref[...] * 0.01
    o_row = lax.broadcasted_iota(jnp.int32, (BB, J, O), 2).astype(f32)
    coef_b = jnp.where(n == o_row, shs_ref[...], 0.0)
    w0 = w2_ref[0:1, :][None, None]
    w1 = w2_ref[1:2, :][None, None]
    ops_ref[...] = coef_a[..., None] * w0 + coef_b[..., None] * w1 + pe
    ma_ref[...] = ash_ref[...] * wma_ref[0:1, :][None]


def _tc_embed(proc_times, nof, shs, ash, w2, wma):
    return pl.pallas_call(
        _tc_body,
        grid=(B // BB,),
        in_specs=[
            pl.BlockSpec((BB, J, O), lambda b: (b, 0, 0)),
            pl.BlockSpec((BB, J, 1), lambda b: (b, 0, 0)),
            pl.BlockSpec((BB, J, 1), lambda b: (b, 0, 0)),
            pl.BlockSpec((BB, M, 1), lambda b: (b, 0, 0)),
            pl.BlockSpec((2, D), lambda b: (0, 0)),
            pl.BlockSpec((1, D), lambda b: (0, 0)),
        ],
        out_specs=[
            pl.BlockSpec((BB, J, O, D), lambda b: (b, 0, 0, 0)),
            pl.BlockSpec((BB, M, D), lambda b: (b, 0, 0)),
        ],
        out_shape=[
            jax.ShapeDtypeStruct((B, J, O, D), jnp.float32),
            jax.ShapeDtypeStruct((B, M, D), jnp.float32),
        ],
    )(proc_times, nof, shs, ash, w2, wma)


def kernel(proc_times, time_job_ready, time_ma_ready, next_op, job_next_ma,
           W_ops, W_ma):
    tjr_p = jnp.pad(time_job_ready, ((0, 0), (0, JP - J)),
                    constant_values=_PAD)
    tma_p = jnp.pad(time_ma_ready, ((0, 0), (0, JP - M)),
                    constant_values=_PAD)
    jnm_p = jnp.pad(job_next_ma.astype(jnp.int32), ((0, 0), (0, JP - J)))
    sh_full, ash_full = _sc_prep(tjr_p, tma_p, jnm_p)
    shs = sh_full[:, :J].reshape(B, J, 1)
    ash = ash_full[:, :M].reshape(B, M, 1)
    nof = next_op.astype(jnp.float32).reshape(B, J, 1)
    w2 = W_ops.T
    wma = W_ma.T
    ops_emb, ma_emb = _tc_embed(proc_times, nof, shs, ash, w2, wma)
    return (ops_emb, ma_emb)


# speedup vs baseline: 1.0786x; 1.0748x over previous
import functools

import jax
import jax.numpy as jnp
from jax import lax
from jax.experimental import pallas as pl
from jax.experimental.pallas import tpu as pltpu
from jax.experimental.pallas import tpu_sc as plsc

B, J, O, M, D = 64, 50, 50, 50, 128
JP = 64
_PAD = 1e30


def _sc_body(tjr_hbm, tma_hbm, jnm_hbm, sh_hbm, ash_hbm,
             tjr_v, tma_v, jnm_v, sh_v, ash_v):
    c = lax.axis_index("c")
    s = lax.axis_index("s")
    wid = s * 2 + c
    for r in range(B // 32):
        b = wid * (B // 32) + r
        pltpu.sync_copy(tjr_hbm.at[b], tjr_v)
        pltpu.sync_copy(tma_hbm.at[b], tma_v)
        pltpu.sync_copy(jnm_hbm.at[b], jnm_v)
        scheds = []
        tmas = []
        for ch in range(JP // 16):
            idx = jnm_v[pl.ds(ch * 16, 16)]
            g = plsc.load_gather(tma_v, [idx])
            scheds.append(jnp.maximum(tjr_v[pl.ds(ch * 16, 16)], g))
            tmas.append(tma_v[pl.ds(ch * 16, 16)])
        m1 = jnp.minimum(jnp.minimum(scheds[0], scheds[1]),
                         jnp.minimum(scheds[2], scheds[3]))
        mn1 = jnp.min(m1)
        m2 = jnp.minimum(jnp.minimum(tmas[0], tmas[1]),
                         jnp.minimum(tmas[2], tmas[3]))
        mn2 = jnp.min(m2)
        for ch in range(JP // 16):
            sh_v[pl.ds(ch * 16, 16)] = (scheds[ch] - mn1) * 0.01
            ash_v[pl.ds(ch * 16, 16)] = tmas[ch] - mn2
        pltpu.sync_copy(sh_v, sh_hbm.at[b])
        pltpu.sync_copy(ash_v, ash_hbm.at[b])


def _sc_prep(tjr_p, tma_p, jnm_p):
    mesh = plsc.VectorSubcoreMesh(core_axis_name="c", subcore_axis_name="s")
    fn = functools.partial(
        pl.kernel,
        out_type=[jax.ShapeDtypeStruct((B, JP), jnp.float32),
                  jax.ShapeDtypeStruct((B, JP), jnp.float32)],
        mesh=mesh,
        compiler_params=pltpu.CompilerParams(needs_layout_passes=False),
        scratch_types=[
            pltpu.VMEM((JP,), jnp.float32),
            pltpu.VMEM((JP,), jnp.float32),
            pltpu.VMEM((JP,), jnp.int32),
            pltpu.VMEM((JP,), jnp.float32),
            pltpu.VMEM((JP,), jnp.float32),
        ],
    )(_sc_body)
    return fn(tjr_p, tma_p, jnm_p)


BB = 4


def _tc_body(proc_ref, nof_ref, noi_ref, shs_ref, ash_ref, w2_ref, wma_ref,
             ops_ref, ma_ref):
    f32 = jnp.float32
    d = lax.broadcasted_iota(jnp.int32, (1, D), 1).astype(f32)
    half = jnp.float32(D // 2)
    dm = jnp.where(d < half, d, d - half)
    freq = jnp.exp(dm * jnp.float32(-jnp.log(10000.0) / (D // 2)))
    phase = jnp.where(d < half, 0.0, jnp.pi / 2).astype(f32)
    o_col = lax.broadcasted_iota(jnp.int32, (O, 1), 0).astype(f32)
    ang_o = o_col * freq
    s_o = jnp.sin(ang_o)[None, None]
    c_o = jnp.cos(ang_o)[None, None]
    n = nof_ref[...]
    ang_n = n * freq[None] + phase[None]
    s_n = jnp.sin(ang_n)[:, :, None]
    c_n = jnp.cos(ang_n)[:, :, None]
    pe = s_o * c_n + c_o * s_n
    coef_a = proc_ref[...] * 0.01
    w0 = w2_ref[0:1, :][None, None]
    w1 = w2_ref[1:2, :]
    r_row = shs_ref[...] * w1[None]
    o4 = lax.broadcasted_iota(jnp.int32, (BB, J, O, D), 2)
    n4 = noi_ref[...][..., None]
    term2 = jnp.where(o4 == n4, r_row[:, :, None, :], 0.0)
    ops_ref[...] = coef_a[..., None] * w0 + pe + term2
    ma_ref[...] = ash_ref[...] * wma_ref[0:1, :][None]


def _tc_embed(proc_times, nof, noi, shs, ash, w2, wma):
    return pl.pallas_call(
        _tc_body,
        grid=(B // BB,),
        in_specs=[
            pl.BlockSpec((BB, J, O), lambda b: (b, 0, 0)),
            pl.BlockSpec((BB, J, 1), lambda b: (b, 0, 0)),
            pl.BlockSpec((BB, J, 1), lambda b: (b, 0, 0)),
            pl.BlockSpec((BB, J, 1), lambda b: (b, 0, 0)),
            pl.BlockSpec((BB, M, 1), lambda b: (b, 0, 0)),
            pl.BlockSpec((2, D), lambda b: (0, 0)),
            pl.BlockSpec((1, D), lambda b: (0, 0)),
        ],
        out_specs=[
            pl.BlockSpec((BB, J, O, D), lambda b: (b, 0, 0, 0)),
            pl.BlockSpec((BB, M, D), lambda b: (b, 0, 0)),
        ],
        out_shape=[
            jax.ShapeDtypeStruct((B, J, O, D), jnp.float32),
            jax.ShapeDtypeStruct((B, M, D), jnp.float32),
        ],
    )(proc_times, nof, noi, shs, ash, w2, wma)


def kernel(proc_times, time_job_ready, time_ma_ready, next_op, job_next_ma,
           W_ops, W_ma):
    tjr_p = jnp.pad(time_job_ready, ((0, 0), (0, JP - J)),
                    constant_values=_PAD)
    tma_p = jnp.pad(time_ma_ready, ((0, 0), (0, JP - M)),
                    constant_values=_PAD)
    jnm_p = jnp.pad(job_next_ma.astype(jnp.int32), ((0, 0), (0, JP - J)))
    sh_full, ash_full = _sc_prep(tjr_p, tma_p, jnm_p)
    shs = sh_full[:, :J].reshape(B, J, 1)
    ash = ash_full[:, :M].reshape(B, M, 1)
    nof = next_op.astype(jnp.float32).reshape(B, J, 1)
    noi = next_op.astype(jnp.int32).reshape(B, J, 1)
    w2 = W_ops.T
    wma = W_ma.T
    ops_emb, ma_emb = _tc_embed(proc_times, nof, noi, shs, ash, w2, wma)
    return (ops_emb, ma_emb)
